# baseline (speedup 1.0000x reference)
import functools

import jax
import jax.numpy as jnp
from jax.experimental import pallas as pl
from jax.experimental.pallas import tpu as pltpu


def _fused_kernel(xa_ref, xb_ref, w1_ref, b1_ref, w2_ref, b2_ref, v_ref,
                  r_ref, c_ref, oa_ref, ob_ref, *, d2, nnz):
    iota = jax.lax.broadcasted_iota(jnp.int32, (d2, nnz), 0)
    rt = (r_ref[0:1, :] == iota).astype(jnp.float32)
    ct = (c_ref[0:1, :] == iota).astype(jnp.float32)
    st = jax.lax.dot_general(
        ct * v_ref[0:1, :], rt,
        (((1,), (1,)), ((), ())),
        preferred_element_type=jnp.float32)
    w12 = jnp.dot(w1_ref[...], w2_ref[...],
                  preferred_element_type=jnp.float32)
    wf = jnp.dot(w12, st, preferred_element_type=jnp.float32)
    bvec = jnp.dot(b1_ref[...], w2_ref[...],
                   preferred_element_type=jnp.float32) + b2_ref[...]
    bf_col = jax.lax.dot_general(
        st, bvec,
        (((0,), (1,)), ((), ())),
        preferred_element_type=jnp.float32)

    for x_ref, o_ref in ((xa_ref, oa_ref), (xb_ref, ob_ref)):
        y_t = jax.lax.dot_general(
            wf, x_ref[...],
            (((0,), (1,)), ((), ())),
            preferred_element_type=jnp.float32)
        o_ref[...] = y_t + bf_col


@jax.jit
def kernel(inputs, W1, b1, W2, b2, sp_vals, sp_rows, sp_cols):
    batch, d_in = inputs.shape
    d1 = W1.shape[1]
    d2 = W2.shape[1]
    nnz = sp_vals.shape[0]

    blk = 4096
    half_steps = batch // (2 * blk)
    grid = (half_steps,)

    full = lambda shape: pl.BlockSpec(shape, lambda i: (0, 0))
    oa, ob = pl.pallas_call(
        functools.partial(_fused_kernel, d2=d2, nnz=nnz),
        grid=grid,
        in_specs=[
            pl.BlockSpec((blk, d_in), lambda i: (i, 0)),
            pl.BlockSpec((blk, d_in),
                         lambda i, hs=half_steps: (i + hs, 0)),
            full((d_in, d1)),
            full((1, d1)),
            full((d1, d2)),
            full((1, d2)),
            full((1, nnz)),
            full((1, nnz)),
            full((1, nnz)),
        ],
        out_specs=[
            pl.BlockSpec((d2, blk), lambda i: (0, i)),
            pl.BlockSpec((d2, blk), lambda i: (0, i)),
        ],
        out_shape=[
            jax.ShapeDtypeStruct((d2, batch // 2), jnp.float32),
            jax.ShapeDtypeStruct((d2, batch // 2), jnp.float32),
        ],
        compiler_params=pltpu.CompilerParams(
            dimension_semantics=("parallel",)),
    )(inputs, inputs, W1, b1.reshape(1, d1), W2, b2.reshape(1, d2),
      sp_vals.reshape(1, nnz), sp_rows.reshape(1, nnz),
      sp_cols.reshape(1, nnz))
    return jnp.concatenate([oa, ob], axis=1).T


# speedup vs baseline: 1.0300x; 1.0300x over previous
import functools

import jax
import jax.numpy as jnp
from jax.experimental import pallas as pl
from jax.experimental.pallas import tpu as pltpu


def _fused_kernel(x_hbm, w1_ref, b1_ref, w2_ref, b2_ref, v_ref, r_ref,
                  c_ref, out_hbm, xbuf, ybuf, in_sem, out_sem, *,
                  d2, nnz, chunk, n_chunks):
    def in_copy(k):
        return pltpu.make_async_copy(
            x_hbm.at[pl.ds(k * chunk, chunk), :],
            xbuf.at[k % 2],
            in_sem.at[k % 2])

    def out_copy(k):
        return pltpu.make_async_copy(
            ybuf.at[k % 2],
            out_hbm.at[:, pl.ds(k * chunk, chunk)],
            out_sem.at[k % 2])

    for k in range(min(2, n_chunks)):
        in_copy(k).start()

    iota = jax.lax.broadcasted_iota(jnp.int32, (d2, nnz), 0)
    rt = (r_ref[0:1, :] == iota).astype(jnp.float32)
    ct = (c_ref[0:1, :] == iota).astype(jnp.float32)
    st = jax.lax.dot_general(
        ct * v_ref[0:1, :], rt,
        (((1,), (1,)), ((), ())),
        preferred_element_type=jnp.float32)
    w12 = jnp.dot(w1_ref[...], w2_ref[...],
                  preferred_element_type=jnp.float32)
    wf = jnp.dot(w12, st, preferred_element_type=jnp.float32)
    bvec = jnp.dot(b1_ref[...], w2_ref[...],
                   preferred_element_type=jnp.float32) + b2_ref[...]
    bf_col = jax.lax.dot_general(
        st, bvec,
        (((0,), (1,)), ((), ())),
        preferred_element_type=jnp.float32)

    for k in range(n_chunks):
        in_copy(k).wait()
        y_t = jax.lax.dot_general(
            wf, xbuf[k % 2],
            (((0,), (1,)), ((), ())),
            preferred_element_type=jnp.float32)
        if k >= 2:
            out_copy(k - 2).wait()
        ybuf[k % 2] = y_t + bf_col
        out_copy(k).start()
        if k + 2 < n_chunks:
            in_copy(k + 2).start()

    for k in range(max(0, n_chunks - 2), n_chunks):
        out_copy(k).wait()


@jax.jit
def kernel(inputs, W1, b1, W2, b2, sp_vals, sp_rows, sp_cols):
    batch, d_in = inputs.shape
    d1 = W1.shape[1]
    d2 = W2.shape[1]
    nnz = sp_vals.shape[0]

    chunk = 2048
    n_chunks = batch // chunk

    full = lambda shape: pl.BlockSpec(shape, lambda: (0, 0))
    out = pl.pallas_call(
        functools.partial(_fused_kernel, d2=d2, nnz=nnz, chunk=chunk,
                          n_chunks=n_chunks),
        in_specs=[
            pl.BlockSpec(memory_space=pltpu.MemorySpace.HBM),
            full((d_in, d1)),
            full((1, d1)),
            full((d1, d2)),
            full((1, d2)),
            full((1, nnz)),
            full((1, nnz)),
            full((1, nnz)),
        ],
        out_specs=pl.BlockSpec(memory_space=pltpu.MemorySpace.HBM),
        out_shape=jax.ShapeDtypeStruct((d2, batch), jnp.float32),
        scratch_shapes=[
            pltpu.VMEM((2, chunk, d_in), jnp.float32),
            pltpu.VMEM((2, d2, chunk), jnp.float32),
            pltpu.SemaphoreType.DMA((2,)),
            pltpu.SemaphoreType.DMA((2,)),
        ],
    )(inputs, W1, b1.reshape(1, d1), W2, b2.reshape(1, d2),
      sp_vals.reshape(1, nnz), sp_rows.reshape(1, nnz),
      sp_cols.reshape(1, nnz))
    return out.T


# speedup vs baseline: 1.3731x; 1.3332x over previous
import functools

import jax
import jax.numpy as jnp
from jax.experimental import pallas as pl
from jax.experimental.pallas import tpu as pltpu


def _fused_kernel(x_hbm, w1_ref, b1_ref, w2_ref, b2_ref, v_ref, r_ref,
                  c_ref, out_hbm, xbuf, ybuf, in_sem, out_sem, *,
                  d2, nnz, chunk, n_chunks):
    def in_copy(k):
        return pltpu.make_async_copy(
            x_hbm.at[pl.ds(k * chunk, chunk), :],
            xbuf.at[k],
            in_sem.at[k])

    def out_copy(k):
        return pltpu.make_async_copy(
            ybuf.at[k],
            out_hbm.at[:, pl.ds(k * chunk, chunk)],
            out_sem.at[k])

    for k in range(n_chunks):
        in_copy(k).start()

    iota = jax.lax.broadcasted_iota(jnp.int32, (d2, nnz), 0)
    rt = (r_ref[0:1, :] == iota).astype(jnp.float32)
    ct = (c_ref[0:1, :] == iota).astype(jnp.float32)
    st = jax.lax.dot_general(
        ct * v_ref[0:1, :], rt,
        (((1,), (1,)), ((), ())),
        preferred_element_type=jnp.float32)
    w12 = jnp.dot(w1_ref[...], w2_ref[...],
                  preferred_element_type=jnp.float32)
    wf = jnp.dot(w12, st, preferred_element_type=jnp.float32)
    bvec = jnp.dot(b1_ref[...], w2_ref[...],
                   preferred_element_type=jnp.float32) + b2_ref[...]
    bf_col = jax.lax.dot_general(
        st, bvec,
        (((0,), (1,)), ((), ())),
        preferred_element_type=jnp.float32)

    for k in range(n_chunks):
        in_copy(k).wait()
        y_t = jax.lax.dot_general(
            wf, xbuf[k],
            (((0,), (1,)), ((), ())),
            preferred_element_type=jnp.float32)
        ybuf[k] = y_t + bf_col
        out_copy(k).start()

    for k in range(n_chunks):
        out_copy(k).wait()


@jax.jit
def kernel(inputs, W1, b1, W2, b2, sp_vals, sp_rows, sp_cols):
    batch, d_in = inputs.shape
    d1 = W1.shape[1]
    d2 = W2.shape[1]
    nnz = sp_vals.shape[0]

    chunk = 2048
    n_chunks = batch // chunk

    full = lambda shape: pl.BlockSpec(shape, lambda: (0, 0))
    out = pl.pallas_call(
        functools.partial(_fused_kernel, d2=d2, nnz=nnz, chunk=chunk,
                          n_chunks=n_chunks),
        in_specs=[
            pl.BlockSpec(memory_space=pltpu.MemorySpace.HBM),
            full((d_in, d1)),
            full((1, d1)),
            full((d1, d2)),
            full((1, d2)),
            full((1, nnz)),
            full((1, nnz)),
            full((1, nnz)),
        ],
        out_specs=pl.BlockSpec(memory_space=pltpu.MemorySpace.HBM),
        out_shape=jax.ShapeDtypeStruct((d2, batch), jnp.float32),
        scratch_shapes=[
            pltpu.VMEM((n_chunks, chunk, d_in), jnp.float32),
            pltpu.VMEM((n_chunks, d2, chunk), jnp.float32),
            pltpu.SemaphoreType.DMA((n_chunks,)),
            pltpu.SemaphoreType.DMA((n_chunks,)),
        ],
    )(inputs, W1, b1.reshape(1, d1), W2, b2.reshape(1, d2),
      sp_vals.reshape(1, nnz), sp_rows.reshape(1, nnz),
      sp_cols.reshape(1, nnz))
    return out.T
